# baseline (speedup 1.0000x reference)
import functools

import jax
import jax.numpy as jnp
from jax import lax
from jax.experimental import pallas as pl
from jax.experimental.pallas import tpu as pltpu
from jax.experimental.pallas import tpu_sc as plsc

N = 2048
E = 32768
S = 512
F32 = jnp.float32


def _build_a1pp(edge_index):
    CHUNK = 8192
    NCHUNK = E // CHUNK
    GROUPS = CHUNK // 16
    mesh = plsc.VectorSubcoreMesh(core_axis_name="c", subcore_axis_name="s")

    @functools.partial(
        pl.kernel,
        mesh=mesh,
        compiler_params=pltpu.CompilerParams(needs_layout_passes=False),
        out_type=jax.ShapeDtypeStruct((16, S * S), F32),
        scratch_types=[
            pltpu.VMEM((CHUNK,), jnp.int32),
            pltpu.VMEM((CHUNK,), jnp.int32),
            pltpu.VMEM((16 * 8 * 512,), F32),
        ],
    )
    def k(edges_hbm, out_hbm, src_v, dst_v, acc):
        wid = lax.axis_index("s") * 2 + lax.axis_index("c")
        zeros16 = jnp.zeros((16,), F32)

        for half in range(2):
            lo = wid * 64 + half * 32
            hi = lo + 32

            def zero_body(i, _):
                acc[pl.ds(i * 16, 16)] = zeros16
                return 0
            lax.fori_loop(0, (16 * 8 * 512) // 16, zero_body, 0)

            for ch in range(NCHUNK):
                pltpu.sync_copy(edges_hbm.at[0, pl.ds(ch * CHUNK, CHUNK)], src_v)
                pltpu.sync_copy(edges_hbm.at[1, pl.ds(ch * CHUNK, CHUNK)], dst_v)

                def group_body(g, _):
                    s16 = src_v[pl.ds(g * 16, 16)]
                    d16 = dst_v[pl.ds(g * 16, 16)]
                    mask = (s16 >= lo) & (s16 < hi)
                    plane = (s16 & 3) * 4 + (d16 & 3)
                    addr = plane * 4096 + ((s16 - lo) >> 2) * 512 + (d16 >> 2)
                    lane = lax.iota(jnp.int32, 16)
                    ones = jnp.full((16,), 1.0, F32)
                    for kk in range(16):
                        plsc.addupdate_scatter(
                            acc.at[pl.ds(0, 16 * 8 * 512)], [addr],
                            ones, mask=mask & (lane == kk))
                    return 0
                lax.fori_loop(0, GROUPS, group_body, 0)

            base = lo >> 2
            for p in range(16):
                pltpu.sync_copy(
                    acc.at[pl.ds(p * 4096, 4096)],
                    out_hbm.at[p, pl.ds(base * 512, 4096)],
                )

    return k(edge_index)


def _shift(b, dr, dc):
    if dr == -1:
        b = jnp.concatenate([jnp.zeros((1, b.shape[1]), b.dtype), b[:-1]], axis=0)
    elif dr == 1:
        b = jnp.concatenate([b[1:], jnp.zeros((1, b.shape[1]), b.dtype)], axis=0)
    if dc == -1:
        b = jnp.concatenate([jnp.zeros((b.shape[0], 1), b.dtype), b[:, :-1]], axis=1)
    elif dc == 1:
        b = jnp.concatenate([b[:, 1:], jnp.zeros((b.shape[0], 1), b.dtype)], axis=1)
    return b


def _k2_body(a_ref, w_ref, b_ref, out_ref):
    og = pl.program_id(0)
    for oc in range(2):
        bias = b_ref[og * 2 + oc]
        for ipp in range(2):
            for jpp in range(2):
                mx = None
                for di in range(2):
                    for dj in range(2):
                        ip = 2 * ipp + di
                        jp = 2 * jpp + dj
                        acc = jnp.zeros((S, S), F32)
                        for a in (-1, 0, 1):
                            q, rI = (ip + a) % 4, (ip + a) // 4
                            for b in (-1, 0, 1):
                                r, rJ = (jp + b) % 4, (jp + b) // 4
                                w = w_ref[og * 2 + oc, a + 1, b + 1]
                                blk = _shift(a_ref[q * 4 + r], rI, rJ)
                                acc = acc + w * blk
                        v = jax.nn.relu(acc + bias)
                        mx = v if mx is None else jnp.maximum(mx, v)
                out_ref[oc, ipp, jpp] = mx


def _k3_body(p1_ref, w_ref, out_ref):
    c = pl.program_id(0)

    @pl.when(c == 0)
    def _():
        out_ref[...] = jnp.zeros_like(out_ref)

    for ip in range(2):
        for jp in range(2):
            blks = {}
            for a in (-1, 0, 1):
                q, rI = (ip + a) % 2, (ip + a) // 2
                for b in (-1, 0, 1):
                    r, rJ = (jp + b) % 2, (jp + b) // 2
                    blks[(a, b)] = _shift(p1_ref[0, q, r], rI, rJ)
            for o2 in range(4):
                acc = out_ref[o2, ip, jp]
                for a in (-1, 0, 1):
                    for b in (-1, 0, 1):
                        acc = acc + w_ref[o2, c, a + 1, b + 1] * blks[(a, b)]
                out_ref[o2, ip, jp] = acc


def _k3b_body(c2_ref, b_ref, p2_ref):
    o2 = pl.program_id(0)
    m = jnp.maximum(
        jnp.maximum(c2_ref[0, 0, 0], c2_ref[0, 0, 1]),
        jnp.maximum(c2_ref[0, 1, 0], c2_ref[0, 1, 1]),
    )
    p2_ref[0] = jax.nn.relu(m + b_ref[o2])


def _k4_body(a1_ref, p2_ref, t1w_ref, t1b_ref, t2w_ref, t2b_ref, out_ref):
    pp = pl.program_id(0)
    ip, jp = pp // 4, pp % 4
    pa, qa = ip // 2, ip % 2
    pb, qb = jp // 2, jp % 2
    acc = [None, None, None]
    for o in range(16):
        u = jnp.zeros((S, S), F32)
        for c in range(4):
            u = u + t1w_ref[c, o, pa, pb] * p2_ref[c]
        u = jax.nn.relu(u + t1b_ref[o])
        for d in range(3):
            t = u * t2w_ref[o, d, qa, qb]
            acc[d] = t if acc[d] is None else acc[d] + t
    a1 = a1_ref[0]
    for d in range(3):
        out_ref[d, 0] = a1 * jax.nn.sigmoid(acc[d] + t2b_ref[d])


def _k5_body(a0_ref, a1_ref, out_ref):
    acc = jnp.zeros((S, S), F32)
    for kp in range(4):
        acc = acc + jnp.dot(a0_ref[0, kp].astype(jnp.bfloat16),
                            a1_ref[kp, 0].astype(jnp.bfloat16),
                            preferred_element_type=F32)
    out_ref[0, 0] = acc


def _k6_body(b_ref, a2_ref, am_ref, parts_ref):
    ip = pl.program_id(0)
    jp = pl.program_id(1)
    acc = jnp.zeros((S, S), F32)
    for kp in range(4):
        acc = acc + jnp.dot(b_ref[0, kp].astype(jnp.bfloat16),
                            a2_ref[kp, 0].astype(jnp.bfloat16),
                            preferred_element_type=F32)
    pos = acc > 0.0
    am = jnp.where(pos, jnp.exp(jnp.log(jnp.where(pos, acc, 1.0)) * (1.0 / 3.0)), 0.0)
    rows = lax.broadcasted_iota(jnp.int32, (S, S), 0)
    cols = lax.broadcasted_iota(jnp.int32, (S, S), 1)
    eye = (rows == cols) & (ip == jp)
    am = jnp.where(eye & (am == 0.0), 1.0, am)
    am_ref[0, 0] = am
    parts_ref[0, 0] = jnp.sum(am, axis=0)


def _dinv_from_parts(parts):
    dinv = []
    for jp in range(4):
        deg = parts[jp * 4 + 0, 0] + parts[jp * 4 + 1, 0] + \
              parts[jp * 4 + 2, 0] + parts[jp * 4 + 3, 0]
        dinv.append(jnp.where(deg > 0, jax.lax.rsqrt(jnp.where(deg > 0, deg, 1.0)), 0.0))
    return dinv


def _select_row(rows, idx):
    out = jnp.zeros((S,), F32)
    for j in range(4):
        out = out + jnp.where(idx == j, 1.0, 0.0) * rows[j]
    return out


def _k7_body(am_ref, parts_ref, xp_ref, w_ref, b_ref, out_ref):
    ip = pl.program_id(0)
    dinv = _dinv_from_parts(parts_ref[...])
    acc = jnp.zeros((S, w_ref.shape[1]), F32)
    for jp in range(4):
        z = dinv[jp][:, None] * jnp.dot(xp_ref[jp], w_ref[...],
                                        preferred_element_type=F32)
        acc = acc + lax.dot_general(am_ref[jp, 0], z,
                                    (((0,), (0,)), ((), ())),
                                    preferred_element_type=F32)
    dout = _select_row(dinv, ip)
    out_ref[0] = jax.nn.relu(dout[:, None] * acc + b_ref[...][None, :])


def _k8_body(am_ref, parts_ref, hp_ref, w_ref, b_ref, out_ref):
    ip = pl.program_id(0)
    dinv = _dinv_from_parts(parts_ref[...])
    acc = jnp.zeros((S, w_ref.shape[1]), F32)
    for jp in range(4):
        z = dinv[jp][:, None] * jnp.dot(hp_ref[jp], w_ref[...],
                                        preferred_element_type=F32)
        acc = acc + lax.dot_general(am_ref[jp, 0], z,
                                    (((0,), (0,)), ((), ())),
                                    preferred_element_type=F32)
    dout = _select_row(dinv, ip)
    o = dout[:, None] * acc + b_ref[...][None, :]
    m = jnp.max(o, axis=1, keepdims=True)
    s = jnp.log(jnp.sum(jnp.exp(o - m), axis=1, keepdims=True))
    out_ref[0] = o - m - s


def kernel(x, c1w, c1b, c2w, c2b, t1w, t1b, t2w, t2b,
           W11, b11, W13, b13, W23, b23, edge_index):
    del W11, b11

    a1pp = _build_a1pp(edge_index).reshape(16, S, S)

    w1e = c1w.sum(axis=1)

    p1pp = pl.pallas_call(
        _k2_body,
        grid=(8,),
        in_specs=[
            pl.BlockSpec((16, S, S), lambda og: (0, 0, 0)),
            pl.BlockSpec(memory_space=pltpu.SMEM),
            pl.BlockSpec(memory_space=pltpu.SMEM),
        ],
        out_specs=pl.BlockSpec((2, 2, 2, S, S), lambda og: (og, 0, 0, 0, 0)),
        out_shape=jax.ShapeDtypeStruct((16, 2, 2, S, S), F32),
    )(a1pp, w1e, c1b)

    c2pp = pl.pallas_call(
        _k3_body,
        grid=(16,),
        in_specs=[
            pl.BlockSpec((1, 2, 2, S, S), lambda c: (c, 0, 0, 0, 0)),
            pl.BlockSpec(memory_space=pltpu.SMEM),
        ],
        out_specs=pl.BlockSpec((4, 2, 2, S, S), lambda c: (0, 0, 0, 0, 0)),
        out_shape=jax.ShapeDtypeStruct((4, 2, 2, S, S), F32),
    )(p1pp, c2w)

    p2 = pl.pallas_call(
        _k3b_body,
        grid=(4,),
        in_specs=[
            pl.BlockSpec((1, 2, 2, S, S), lambda o: (o, 0, 0, 0, 0)),
            pl.BlockSpec(memory_space=pltpu.SMEM),
        ],
        out_specs=pl.BlockSpec((1, S, S), lambda o: (o, 0, 0)),
        out_shape=jax.ShapeDtypeStruct((4, S, S), F32),
    )(c2pp, c2b)

    app = pl.pallas_call(
        _k4_body,
        grid=(16,),
        in_specs=[
            pl.BlockSpec((1, S, S), lambda pp: (pp, 0, 0)),
            pl.BlockSpec((4, S, S), lambda pp: (0, 0, 0)),
            pl.BlockSpec(memory_space=pltpu.SMEM),
            pl.BlockSpec(memory_space=pltpu.SMEM),
            pl.BlockSpec(memory_space=pltpu.SMEM),
            pl.BlockSpec(memory_space=pltpu.SMEM),
        ],
        out_specs=pl.BlockSpec((3, 1, S, S), lambda pp: (0, pp, 0, 0)),
        out_shape=jax.ShapeDtypeStruct((3, 16, S, S), F32),
    )(a1pp, p2, t1w, t1b, t2w, t2b)

    app = app.reshape(3, 4, 4, S, S)

    bpp = pl.pallas_call(
        _k5_body,
        grid=(4, 4),
        in_specs=[
            pl.BlockSpec((1, 4, S, S), lambda i, j: (i, 0, 0, 0)),
            pl.BlockSpec((4, 1, S, S), lambda i, j: (0, j, 0, 0)),
        ],
        out_specs=pl.BlockSpec((1, 1, S, S), lambda i, j: (i, j, 0, 0)),
        out_shape=jax.ShapeDtypeStruct((4, 4, S, S), F32),
    )(app[0], app[1])

    am, parts = pl.pallas_call(
        _k6_body,
        grid=(4, 4),
        in_specs=[
            pl.BlockSpec((1, 4, S, S), lambda i, j: (i, 0, 0, 0)),
            pl.BlockSpec((4, 1, S, S), lambda i, j: (0, j, 0, 0)),
        ],
        out_specs=[
            pl.BlockSpec((1, 1, S, S), lambda i, j: (i, j, 0, 0)),
            pl.BlockSpec((1, 1, S), lambda i, j: (j * 4 + i, 0, 0)),
        ],
        out_shape=[
            jax.ShapeDtypeStruct((4, 4, S, S), F32),
            jax.ShapeDtypeStruct((16, 1, S), F32),
        ],
    )(bpp, app[2])

    xp = x.reshape(S, 4, x.shape[1]).transpose(1, 0, 2)

    hp = pl.pallas_call(
        _k7_body,
        grid=(4,),
        in_specs=[
            pl.BlockSpec((4, 1, S, S), lambda i: (0, i, 0, 0)),
            pl.BlockSpec((16, 1, S), lambda i: (0, 0, 0)),
            pl.BlockSpec((4, S, 128), lambda i: (0, 0, 0)),
            pl.BlockSpec((128, 64), lambda i: (0, 0)),
            pl.BlockSpec((64,), lambda i: (0,)),
        ],
        out_specs=pl.BlockSpec((1, S, 64), lambda i: (i, 0, 0)),
        out_shape=jax.ShapeDtypeStruct((4, S, 64), F32),
    )(am, parts, xp, W13, b13)

    op = pl.pallas_call(
        _k8_body,
        grid=(4,),
        in_specs=[
            pl.BlockSpec((4, 1, S, S), lambda i: (0, i, 0, 0)),
            pl.BlockSpec((16, 1, S), lambda i: (0, 0, 0)),
            pl.BlockSpec((4, S, 64), lambda i: (0, 0, 0)),
            pl.BlockSpec((64, 16), lambda i: (0, 0)),
            pl.BlockSpec((16,), lambda i: (0,)),
        ],
        out_specs=pl.BlockSpec((1, S, 16), lambda i: (i, 0, 0)),
        out_shape=jax.ShapeDtypeStruct((4, S, 16), F32),
    )(am, parts, hp, W23, b23)

    return op.transpose(1, 0, 2).reshape(N, 16)


# speedup vs baseline: 1.1793x; 1.1793x over previous
import functools

import jax
import jax.numpy as jnp
from jax import lax
from jax.experimental import pallas as pl
from jax.experimental.pallas import tpu as pltpu
from jax.experimental.pallas import tpu_sc as plsc

N = 2048
E = 32768
S = 512
F32 = jnp.float32
BF16 = jnp.bfloat16


def _build_a1pp(edge_index):
    CHUNK = 8192
    NCHUNK = E // CHUNK
    GROUPS = CHUNK // 16
    mesh = plsc.VectorSubcoreMesh(core_axis_name="c", subcore_axis_name="s")

    @functools.partial(
        pl.kernel,
        mesh=mesh,
        compiler_params=pltpu.CompilerParams(needs_layout_passes=False),
        out_type=jax.ShapeDtypeStruct((16, S * S), F32),
        scratch_types=[
            pltpu.VMEM((CHUNK,), jnp.int32),
            pltpu.VMEM((CHUNK,), jnp.int32),
            pltpu.VMEM((16 * 8 * 512,), F32),
        ],
    )
    def k(edges_hbm, out_hbm, src_v, dst_v, acc):
        wid = lax.axis_index("s") * 2 + lax.axis_index("c")
        zeros16 = jnp.zeros((16,), F32)

        for half in range(2):
            lo = wid * 64 + half * 32
            hi = lo + 32

            def zero_body(i, _):
                acc[pl.ds(i * 16, 16)] = zeros16
                return 0
            lax.fori_loop(0, (16 * 8 * 512) // 16, zero_body, 0)

            for ch in range(NCHUNK):
                pltpu.sync_copy(edges_hbm.at[0, pl.ds(ch * CHUNK, CHUNK)], src_v)
                pltpu.sync_copy(edges_hbm.at[1, pl.ds(ch * CHUNK, CHUNK)], dst_v)

                def group_body(g, _):
                    s16 = src_v[pl.ds(g * 16, 16)]
                    d16 = dst_v[pl.ds(g * 16, 16)]
                    mask = (s16 >= lo) & (s16 < hi)
                    plane = (s16 & 3) * 4 + (d16 & 3)
                    addr = plane * 4096 + ((s16 - lo) >> 2) * 512 + (d16 >> 2)
                    lane = lax.iota(jnp.int32, 16)
                    ones = jnp.full((16,), 1.0, F32)
                    for kk in range(16):
                        plsc.addupdate_scatter(
                            acc.at[pl.ds(0, 16 * 8 * 512)], [addr],
                            ones, mask=mask & (lane == kk))
                    return 0
                lax.fori_loop(0, GROUPS, group_body, 0)

            base = lo >> 2
            for p in range(16):
                pltpu.sync_copy(
                    acc.at[pl.ds(p * 4096, 4096)],
                    out_hbm.at[p, pl.ds(base * 512, 4096)],
                )

    return k(edge_index)


def _shift2(b, dr, dc):
    z = jnp.zeros_like(b)
    if dr == -1:
        b = jnp.concatenate([z[..., :1, :], b[..., :-1, :]], axis=-2)
    elif dr == 1:
        b = jnp.concatenate([b[..., 1:, :], z[..., :1, :]], axis=-2)
    if dc == -1:
        b = jnp.concatenate([z[..., :, :1], b[..., :, :-1]], axis=-1)
    elif dc == 1:
        b = jnp.concatenate([b[..., :, 1:], z[..., :, :1]], axis=-1)
    return b


def _make_k2_body(ipp, jpp):
    def body(a_ref, w_ref, b_ref, out_ref):
        wmat = w_ref[0].astype(BF16)
        bvec = b_ref[0, 0]
        mx = None
        for di in range(2):
            for dj in range(2):
                ip = 2 * ipp + di
                jp = 2 * jpp + dj
                planes = []
                for a in (-1, 0, 1):
                    q, rI = (ip + a) % 4, (ip + a) // 4
                    for b in (-1, 0, 1):
                        r, rJ = (jp + b) % 4, (jp + b) // 4
                        planes.append(_shift2(a_ref[q * 4 + r], rI, rJ))
                rhs = jnp.stack(planes).reshape(9, S * S)
                res = jnp.dot(wmat, rhs, preferred_element_type=F32)
                v = jax.nn.relu(res + bvec[:, None])
                mx = v if mx is None else jnp.maximum(mx, v)
        out_ref[...] = mx.reshape(8, S, S).astype(BF16)
    return body


def _make_k3_body(ip, jp):
    def body(p00_ref, p01_ref, p10_ref, p11_ref, w_ref, out_ref):
        prefs = ((p00_ref, p01_ref), (p10_ref, p11_ref))
        acc = jnp.zeros((4, S, S), F32)
        for a in (-1, 0, 1):
            q, rI = (ip + a) % 2, (ip + a) // 2
            for b in (-1, 0, 1):
                r, rJ = (jp + b) % 2, (jp + b) // 2
                P = prefs[q][r][...].reshape(8, S * S)
                W = w_ref[:, :, a + 1, b + 1].astype(BF16)
                res = jnp.dot(W, P, preferred_element_type=F32)
                acc = acc + _shift2(res.reshape(4, S, S), rI, rJ)
        out_ref[...] = acc.astype(BF16)
    return body


def _k3b_body(*refs):
    parts = refs[:8]
    bias_ref = refs[8]
    p2_ref = refs[9]
    o2 = pl.program_id(0)
    mx = None
    for par in range(4):
        v = parts[par * 2][0].astype(F32) + parts[par * 2 + 1][0].astype(F32)
        mx = v if mx is None else jnp.maximum(mx, v)
    p2_ref[0] = jax.nn.relu(mx + bias_ref[o2]).astype(BF16)


def _k4_body(a1_ref, p2_ref, t1w_ref, t1b_ref, t2w_ref, t2b_ref, out_ref):
    p2f = p2_ref[...].reshape(4, S * S)
    w1 = t1w_ref[0, 0].astype(BF16)
    u = jnp.dot(w1, p2f, preferred_element_type=F32)
    u = jax.nn.relu(u + t1b_ref[...][:, None]).astype(BF16)
    for qa in range(2):
        for qb in range(2):
            w2 = t2w_ref[qa, qb].astype(BF16)
            att = jnp.dot(w2, u, preferred_element_type=F32)
            att = jax.nn.sigmoid(att + t2b_ref[...][:, None])
            a1 = a1_ref[0, qa, 0, qb].reshape(1, S * S).astype(F32)
            out_ref[:, 0, 0, qa, qb] = (a1 * att).reshape(3, S, S).astype(BF16)


def _k5_body(a0_ref, a1_ref, out_ref):
    acc = jnp.zeros((S, S), F32)
    for kp in range(4):
        acc = acc + jnp.dot(a0_ref[0, kp // 2, 0, kp % 2],
                            a1_ref[kp // 2, 0, kp % 2, 0],
                            preferred_element_type=F32)
    out_ref[0, 0] = acc.astype(BF16)


def _k6_body(b_ref, a2_ref, am_ref, parts_ref):
    ip = pl.program_id(0)
    jp = pl.program_id(1)
    acc = jnp.zeros((S, S), F32)
    for kp in range(4):
        acc = acc + jnp.dot(b_ref[0, kp],
                            a2_ref[kp // 2, 0, kp % 2, 0],
                            preferred_element_type=F32)
    pos = acc > 0.0
    am = jnp.where(pos, jnp.exp(jnp.log(jnp.where(pos, acc, 1.0)) * (1.0 / 3.0)), 0.0)
    rows = lax.broadcasted_iota(jnp.int32, (S, S), 0)
    cols = lax.broadcasted_iota(jnp.int32, (S, S), 1)
    eye = (rows == cols) & (ip == jp)
    am = jnp.where(eye & (am == 0.0), 1.0, am)
    am_ref[0, 0] = am.astype(BF16)
    parts_ref[0, 0] = jnp.sum(am, axis=0)


def _dinv_from_parts(parts):
    dinv = []
    for jp in range(4):
        deg = parts[jp * 4 + 0, 0] + parts[jp * 4 + 1, 0] + \
              parts[jp * 4 + 2, 0] + parts[jp * 4 + 3, 0]
        dinv.append(jnp.where(deg > 0, jax.lax.rsqrt(jnp.where(deg > 0, deg, 1.0)), 0.0))
    return dinv


def _select_row(rows, idx):
    out = jnp.zeros((S,), F32)
    for j in range(4):
        out = out + jnp.where(idx == j, 1.0, 0.0) * rows[j]
    return out


def _k7_body(am_ref, parts_ref, xp_ref, w_ref, b_ref, out_ref):
    ip = pl.program_id(0)
    dinv = _dinv_from_parts(parts_ref[...])
    acc = jnp.zeros((S, w_ref.shape[1]), F32)
    for jp in range(4):
        z = dinv[jp][:, None] * jnp.dot(xp_ref[jp], w_ref[...],
                                        preferred_element_type=F32)
        acc = acc + lax.dot_general(am_ref[jp, 0], z.astype(BF16),
                                    (((0,), (0,)), ((), ())),
                                    preferred_element_type=F32)
    dout = _select_row(dinv, ip)
    out_ref[0] = jax.nn.relu(dout[:, None] * acc + b_ref[...][None, :])


def _k8_body(am_ref, parts_ref, hp_ref, w_ref, b_ref, out_ref):
    ip = pl.program_id(0)
    dinv = _dinv_from_parts(parts_ref[...])
    acc = jnp.zeros((S, w_ref.shape[1]), F32)
    for jp in range(4):
        z = dinv[jp][:, None] * jnp.dot(hp_ref[jp], w_ref[...],
                                        preferred_element_type=F32)
        acc = acc + lax.dot_general(am_ref[jp, 0], z.astype(BF16),
                                    (((0,), (0,)), ((), ())),
                                    preferred_element_type=F32)
    dout = _select_row(dinv, ip)
    o = dout[:, None] * acc + b_ref[...][None, :]
    m = jnp.max(o, axis=1, keepdims=True)
    s = jnp.log(jnp.sum(jnp.exp(o - m), axis=1, keepdims=True))
    out_ref[0] = o - m - s


def kernel(x, c1w, c1b, c2w, c2b, t1w, t1b, t2w, t2b,
           W11, b11, W13, b13, W23, b23, edge_index):
    del W11, b11

    a1pp = _build_a1pp(edge_index).reshape(16, S, S)
    a1b = a1pp.astype(BF16)

    w1e = c1w.sum(axis=1).reshape(2, 8, 9)
    c1br = c1b.reshape(2, 1, 8)

    p1q = []
    for ipp in range(2):
        for jpp in range(2):
            p1q.append(pl.pallas_call(
                _make_k2_body(ipp, jpp),
                grid=(2,),
                in_specs=[
                    pl.BlockSpec((16, S, S), lambda og: (0, 0, 0)),
                    pl.BlockSpec((1, 8, 9), lambda og: (og, 0, 0)),
                    pl.BlockSpec((1, 1, 8), lambda og: (og, 0, 0)),
                ],
                out_specs=pl.BlockSpec((8, S, S), lambda og: (og, 0, 0)),
                out_shape=jax.ShapeDtypeStruct((16, S, S), BF16),
            )(a1b, w1e, c1br))

    c2parts = []
    for ip in range(2):
        for jp in range(2):
            for cg in range(2):
                c2parts.append(pl.pallas_call(
                    _make_k3_body(ip, jp),
                    grid=(1,),
                    in_specs=[
                        pl.BlockSpec((8, S, S), lambda i, cg=cg: (cg, 0, 0)),
                        pl.BlockSpec((8, S, S), lambda i, cg=cg: (cg, 0, 0)),
                        pl.BlockSpec((8, S, S), lambda i, cg=cg: (cg, 0, 0)),
                        pl.BlockSpec((8, S, S), lambda i, cg=cg: (cg, 0, 0)),
                        pl.BlockSpec((4, 8, 3, 3), lambda i, cg=cg: (0, cg, 0, 0)),
                    ],
                    out_specs=pl.BlockSpec((4, S, S), lambda i: (0, 0, 0)),
                    out_shape=jax.ShapeDtypeStruct((4, S, S), BF16),
                )(p1q[0], p1q[1], p1q[2], p1q[3], c2w))

    p2 = pl.pallas_call(
        _k3b_body,
        grid=(4,),
        in_specs=[pl.BlockSpec((1, S, S), lambda o: (o, 0, 0))] * 8
                 + [pl.BlockSpec(memory_space=pltpu.SMEM)],
        out_specs=pl.BlockSpec((1, S, S), lambda o: (o, 0, 0)),
        out_shape=jax.ShapeDtypeStruct((4, S, S), BF16),
    )(*c2parts, c2b)

    a1v = a1b.reshape(2, 2, 2, 2, S, S)
    t1wr = t1w.transpose(2, 3, 1, 0)
    t2wr = t2w.transpose(2, 3, 1, 0)

    app = pl.pallas_call(
        _k4_body,
        grid=(4,),
        in_specs=[
            pl.BlockSpec((1, 2, 1, 2, S, S), lambda g: (g // 2, 0, g % 2, 0, 0, 0)),
            pl.BlockSpec((4, S, S), lambda g: (0, 0, 0)),
            pl.BlockSpec((1, 1, 16, 4), lambda g: (g // 2, g % 2, 0, 0)),
            pl.BlockSpec((16,), lambda g: (0,)),
            pl.BlockSpec((2, 2, 3, 16), lambda g: (0, 0, 0, 0)),
            pl.BlockSpec((3,), lambda g: (0,)),
        ],
        out_specs=pl.BlockSpec((3, 1, 1, 2, 2, S, S),
                               lambda g: (0, g // 2, g % 2, 0, 0, 0, 0)),
        out_shape=jax.ShapeDtypeStruct((3, 2, 2, 2, 2, S, S), BF16),
    )(a1v, p2, t1wr, t1b, t2wr, t2b)

    bpp = pl.pallas_call(
        _k5_body,
        grid=(4, 4),
        in_specs=[
            pl.BlockSpec((1, 2, 1, 2, S, S),
                         lambda i, j: (i // 2, 0, i % 2, 0, 0, 0)),
            pl.BlockSpec((2, 1, 2, 1, S, S),
                         lambda i, j: (0, j // 2, 0, j % 2, 0, 0)),
        ],
        out_specs=pl.BlockSpec((1, 1, S, S), lambda i, j: (i, j, 0, 0)),
        out_shape=jax.ShapeDtypeStruct((4, 4, S, S), BF16),
    )(app[0], app[1])

    am, parts = pl.pallas_call(
        _k6_body,
        grid=(4, 4),
        in_specs=[
            pl.BlockSpec((1, 4, S, S), lambda i, j: (i, 0, 0, 0)),
            pl.BlockSpec((2, 1, 2, 1, S, S),
                         lambda i, j: (0, j // 2, 0, j % 2, 0, 0)),
        ],
        out_specs=[
            pl.BlockSpec((1, 1, S, S), lambda i, j: (i, j, 0, 0)),
            pl.BlockSpec((1, 1, S), lambda i, j: (j * 4 + i, 0, 0)),
        ],
        out_shape=[
            jax.ShapeDtypeStruct((4, 4, S, S), BF16),
            jax.ShapeDtypeStruct((16, 1, S), F32),
        ],
    )(bpp, app[2])

    xp = x.reshape(S, 4, x.shape[1]).transpose(1, 0, 2)

    hp = pl.pallas_call(
        _k7_body,
        grid=(4,),
        in_specs=[
            pl.BlockSpec((4, 1, S, S), lambda i: (0, i, 0, 0)),
            pl.BlockSpec((16, 1, S), lambda i: (0, 0, 0)),
            pl.BlockSpec((4, S, 128), lambda i: (0, 0, 0)),
            pl.BlockSpec((128, 64), lambda i: (0, 0)),
            pl.BlockSpec((64,), lambda i: (0,)),
        ],
        out_specs=pl.BlockSpec((1, S, 64), lambda i: (i, 0, 0)),
        out_shape=jax.ShapeDtypeStruct((4, S, 64), F32),
    )(am, parts, xp, W13, b13)

    op = pl.pallas_call(
        _k8_body,
        grid=(4,),
        in_specs=[
            pl.BlockSpec((4, 1, S, S), lambda i: (0, i, 0, 0)),
            pl.BlockSpec((16, 1, S), lambda i: (0, 0, 0)),
            pl.BlockSpec((4, S, 64), lambda i: (0, 0, 0)),
            pl.BlockSpec((64, 16), lambda i: (0, 0)),
            pl.BlockSpec((16,), lambda i: (0,)),
        ],
        out_specs=pl.BlockSpec((1, S, 16), lambda i: (i, 0, 0)),
        out_shape=jax.ShapeDtypeStruct((4, S, 16), F32),
    )(am, parts, hp, W23, b23)

    return op.transpose(1, 0, 2).reshape(N, 16)


# speedup vs baseline: 1.3182x; 1.1178x over previous
import functools

import jax
import jax.numpy as jnp
from jax import lax
from jax.experimental import pallas as pl
from jax.experimental.pallas import tpu as pltpu
from jax.experimental.pallas import tpu_sc as plsc

N = 2048
E = 32768
S = 512
F32 = jnp.float32
BF16 = jnp.bfloat16


def _build_a1pp(edge_index):
    CHUNK = 8192
    NCHUNK = E // CHUNK
    GROUPS = CHUNK // 16
    mesh = plsc.VectorSubcoreMesh(core_axis_name="c", subcore_axis_name="s")

    @functools.partial(
        pl.kernel,
        mesh=mesh,
        compiler_params=pltpu.CompilerParams(needs_layout_passes=False),
        out_type=jax.ShapeDtypeStruct((16, S * S), F32),
        scratch_types=[
            pltpu.VMEM((CHUNK,), jnp.int32),
            pltpu.VMEM((CHUNK,), jnp.int32),
            pltpu.VMEM((16 * 8 * 512,), F32),
        ],
    )
    def k(edges_hbm, out_hbm, src_v, dst_v, acc):
        wid = lax.axis_index("s") * 2 + lax.axis_index("c")
        zeros16 = jnp.zeros((16,), F32)

        for half in range(2):
            lo = wid * 64 + half * 32
            hi = lo + 32

            def zero_body(i, _):
                acc[pl.ds(i * 16, 16)] = zeros16
                return 0
            lax.fori_loop(0, (16 * 8 * 512) // 16, zero_body, 0)

            for ch in range(NCHUNK):
                pltpu.sync_copy(edges_hbm.at[0, pl.ds(ch * CHUNK, CHUNK)], src_v)
                pltpu.sync_copy(edges_hbm.at[1, pl.ds(ch * CHUNK, CHUNK)], dst_v)

                def group_body(g, _):
                    s16 = src_v[pl.ds(g * 16, 16)]
                    d16 = dst_v[pl.ds(g * 16, 16)]
                    mask = (s16 >= lo) & (s16 < hi)
                    plane = (s16 & 3) * 4 + (d16 & 3)
                    addr = plane * 4096 + ((s16 - lo) >> 2) * 512 + (d16 >> 2)
                    lane = lax.iota(jnp.int32, 16)
                    ones = jnp.full((16,), 1.0, F32)
                    for kk in range(16):
                        plsc.addupdate_scatter(
                            acc.at[pl.ds(0, 16 * 8 * 512)], [addr],
                            ones, mask=mask & (lane == kk))
                    return 0
                lax.fori_loop(0, GROUPS, group_body, 0)

            base = lo >> 2
            for p in range(16):
                pltpu.sync_copy(
                    acc.at[pl.ds(p * 4096, 4096)],
                    out_hbm.at[p, pl.ds(base * 512, 4096)],
                )

    return k(edge_index)


def _build_a1pp_v2(edge_index):
    EPT = E // 16
    G = EPT // 16
    QW = 16 * 128 * 512
    mesh = plsc.VectorSubcoreMesh(core_axis_name="c", subcore_axis_name="s")

    @functools.partial(
        pl.kernel,
        mesh=mesh,
        compiler_params=pltpu.CompilerParams(needs_layout_passes=False),
        out_type=jax.ShapeDtypeStruct((16, S * S), F32),
        scratch_types=[
            pltpu.VMEM((EPT,), jnp.int32),
            pltpu.VMEM((EPT,), jnp.int32),
            pltpu.VMEM((2, 16, 128), jnp.int32),
            pltpu.VMEM((2, 16, 128), F32),
            pltpu.VMEM((16384,), F32),
            pltpu.VMEM_SHARED((QW + 8,), F32),
        ],
    )
    def k(edges_hbm, out_hbm, src_v, dst_v, idx_st, val_st, zbuf, shared):
        sc = lax.axis_index("c")
        tile = lax.axis_index("s")
        base_e = tile * EPT
        pltpu.sync_copy(edges_hbm.at[0, pl.ds(base_e, EPT)], src_v)
        pltpu.sync_copy(edges_hbm.at[1, pl.ds(base_e, EPT)], dst_v)

        zeros16 = jnp.zeros((16,), F32)

        def zb_body(i, _):
            zbuf[pl.ds(i * 16, 16)] = zeros16
            return 0
        lax.fori_loop(0, 16384 // 16, zb_body, 0)

        row0 = sc * 1024

        def gbody(g, _):
            s16 = src_v[pl.ds(g * 16, 16)]
            d16 = dst_v[pl.ds(g * 16, 16)]
            for r in range(2):
                lo = row0 + r * 512
                inq = (s16 >= lo) & (s16 < lo + 512)
                addr = ((s16 & 3) * 4 + (d16 & 3)) * 65536                     + ((s16 - lo) >> 2) * 512 + (d16 >> 2)
                idx_st[r, g >> 3, pl.ds((g & 7) * 16, 16)] =                     jnp.where(inq, addr, QW)
                val_st[r, g >> 3, pl.ds((g & 7) * 16, 16)] =                     jnp.where(inq, 1.0, 0.0)
            return 0
        lax.fori_loop(0, G, gbody, 0)

        for r in range(2):
            for z in range(4):
                pltpu.sync_copy(
                    zbuf, shared.at[pl.ds(tile * 65536 + z * 16384, 16384)])
            plsc.subcore_barrier()
            for j in range(16):
                pltpu.sync_copy(val_st.at[r, j],
                                shared.at[idx_st.at[r, j]], add=True)
            plsc.subcore_barrier()
            q = sc * 2 + r
            pltpu.sync_copy(shared.at[pl.ds(tile * 65536, 65536)],
                            out_hbm.at[tile, pl.ds(q * 65536, 65536)])
            plsc.subcore_barrier()

    return k(edge_index)


def _shift2(b, dr, dc):
    z = jnp.zeros_like(b)
    if dr == -1:
        b = jnp.concatenate([z[..., :1, :], b[..., :-1, :]], axis=-2)
    elif dr == 1:
        b = jnp.concatenate([b[..., 1:, :], z[..., :1, :]], axis=-2)
    if dc == -1:
        b = jnp.concatenate([z[..., :, :1], b[..., :, :-1]], axis=-1)
    elif dc == 1:
        b = jnp.concatenate([b[..., :, 1:], z[..., :, :1]], axis=-1)
    return b


def _make_k2_body(ipp, jpp):
    def body(a_ref, w_ref, b_ref, out_ref):
        wmat = w_ref[0].astype(BF16)
        bvec = b_ref[0, 0]
        mx = None
        for di in range(2):
            for dj in range(2):
                ip = 2 * ipp + di
                jp = 2 * jpp + dj
                planes = []
                for a in (-1, 0, 1):
                    q, rI = (ip + a) % 4, (ip + a) // 4
                    for b in (-1, 0, 1):
                        r, rJ = (jp + b) % 4, (jp + b) // 4
                        planes.append(_shift2(a_ref[q * 4 + r], rI, rJ))
                rhs = jnp.stack(planes).reshape(9, S * S)
                res = jnp.dot(wmat, rhs, preferred_element_type=F32)
                v = jax.nn.relu(res + bvec[:, None])
                mx = v if mx is None else jnp.maximum(mx, v)
        out_ref[...] = mx.reshape(8, S, S).astype(BF16)
    return body


def _make_k3_body(ip, jp):
    def body(p00_ref, p01_ref, p10_ref, p11_ref, w_ref, out_ref):
        prefs = ((p00_ref, p01_ref), (p10_ref, p11_ref))
        acc = jnp.zeros((4, S, S), F32)
        for a in (-1, 0, 1):
            q, rI = (ip + a) % 2, (ip + a) // 2
            for b in (-1, 0, 1):
                r, rJ = (jp + b) % 2, (jp + b) // 2
                P = prefs[q][r][...].reshape(8, S * S)
                W = w_ref[:, :, a + 1, b + 1].astype(BF16)
                res = jnp.dot(W, P, preferred_element_type=F32)
                acc = acc + _shift2(res.reshape(4, S, S), rI, rJ)
        out_ref[...] = acc.astype(BF16)
    return body


def _k3b_body(*refs):
    parts = refs[:8]
    bias_ref = refs[8]
    p2_ref = refs[9]
    o2 = pl.program_id(0)
    mx = None
    for par in range(4):
        v = parts[par * 2][0].astype(F32) + parts[par * 2 + 1][0].astype(F32)
        mx = v if mx is None else jnp.maximum(mx, v)
    p2_ref[0] = jax.nn.relu(mx + bias_ref[o2]).astype(BF16)


def _k4_body(a1_ref, p2_ref, t1w_ref, t1b_ref, t2w_ref, t2b_ref, out_ref):
    p2f = p2_ref[...].reshape(4, S * S)
    w1 = t1w_ref[0, 0].astype(BF16)
    u = jnp.dot(w1, p2f, preferred_element_type=F32)
    u = jax.nn.relu(u + t1b_ref[...][:, None]).astype(BF16)
    for qa in range(2):
        for qb in range(2):
            w2 = t2w_ref[qa, qb].astype(BF16)
            att = jnp.dot(w2, u, preferred_element_type=F32)
            att = jax.nn.sigmoid(att + t2b_ref[...][:, None])
            a1 = a1_ref[0, qa, 0, qb].reshape(1, S * S).astype(F32)
            out_ref[:, 0, 0, qa, qb] = (a1 * att).reshape(3, S, S).astype(BF16)


def _k5_body(a0_ref, a1_ref, out_ref):
    acc = jnp.zeros((S, S), F32)
    for kp in range(4):
        acc = acc + jnp.dot(a0_ref[0, kp // 2, 0, kp % 2],
                            a1_ref[kp // 2, 0, kp % 2, 0],
                            preferred_element_type=F32)
    out_ref[0, 0] = acc.astype(BF16)


def _k6_body(b_ref, a2_ref, am_ref, parts_ref):
    ip = pl.program_id(0)
    jp = pl.program_id(1)
    acc = jnp.zeros((S, S), F32)
    for kp in range(4):
        acc = acc + jnp.dot(b_ref[0, kp],
                            a2_ref[kp // 2, 0, kp % 2, 0],
                            preferred_element_type=F32)
    pos = acc > 0.0
    am = jnp.where(pos, jnp.exp(jnp.log(jnp.where(pos, acc, 1.0)) * (1.0 / 3.0)), 0.0)
    rows = lax.broadcasted_iota(jnp.int32, (S, S), 0)
    cols = lax.broadcasted_iota(jnp.int32, (S, S), 1)
    eye = (rows == cols) & (ip == jp)
    am = jnp.where(eye & (am == 0.0), 1.0, am)
    am_ref[0, 0] = am.astype(BF16)
    parts_ref[0, 0] = jnp.sum(am, axis=0)


def _dinv_from_parts(parts):
    dinv = []
    for jp in range(4):
        deg = parts[jp * 4 + 0, 0] + parts[jp * 4 + 1, 0] + \
              parts[jp * 4 + 2, 0] + parts[jp * 4 + 3, 0]
        dinv.append(jnp.where(deg > 0, jax.lax.rsqrt(jnp.where(deg > 0, deg, 1.0)), 0.0))
    return dinv


def _select_row(rows, idx):
    out = jnp.zeros((S,), F32)
    for j in range(4):
        out = out + jnp.where(idx == j, 1.0, 0.0) * rows[j]
    return out


def _k7_body(am_ref, parts_ref, xp_ref, w_ref, b_ref, out_ref):
    ip = pl.program_id(0)
    dinv = _dinv_from_parts(parts_ref[...])
    acc = jnp.zeros((S, w_ref.shape[1]), F32)
    for jp in range(4):
        z = dinv[jp][:, None] * jnp.dot(xp_ref[jp], w_ref[...],
                                        preferred_element_type=F32)
        acc = acc + lax.dot_general(am_ref[jp, 0], z.astype(BF16),
                                    (((0,), (0,)), ((), ())),
                                    preferred_element_type=F32)
    dout = _select_row(dinv, ip)
    out_ref[0] = jax.nn.relu(dout[:, None] * acc + b_ref[...][None, :])


def _k8_body(am_ref, parts_ref, hp_ref, w_ref, b_ref, out_ref):
    ip = pl.program_id(0)
    dinv = _dinv_from_parts(parts_ref[...])
    acc = jnp.zeros((S, w_ref.shape[1]), F32)
    for jp in range(4):
        z = dinv[jp][:, None] * jnp.dot(hp_ref[jp], w_ref[...],
                                        preferred_element_type=F32)
        acc = acc + lax.dot_general(am_ref[jp, 0], z.astype(BF16),
                                    (((0,), (0,)), ((), ())),
                                    preferred_element_type=F32)
    dout = _select_row(dinv, ip)
    o = dout[:, None] * acc + b_ref[...][None, :]
    m = jnp.max(o, axis=1, keepdims=True)
    s = jnp.log(jnp.sum(jnp.exp(o - m), axis=1, keepdims=True))
    out_ref[0] = o - m - s


def kernel(x, c1w, c1b, c2w, c2b, t1w, t1b, t2w, t2b,
           W11, b11, W13, b13, W23, b23, edge_index):
    del W11, b11

    a1pp = _build_a1pp_v2(edge_index).reshape(16, S, S)
    a1b = a1pp.astype(BF16)

    w1e = c1w.sum(axis=1).reshape(2, 8, 9)
    c1br = c1b.reshape(2, 1, 8)

    p1q = []
    for ipp in range(2):
        for jpp in range(2):
            p1q.append(pl.pallas_call(
                _make_k2_body(ipp, jpp),
                grid=(2,),
                in_specs=[
                    pl.BlockSpec((16, S, S), lambda og: (0, 0, 0)),
                    pl.BlockSpec((1, 8, 9), lambda og: (og, 0, 0)),
                    pl.BlockSpec((1, 1, 8), lambda og: (og, 0, 0)),
                ],
                out_specs=pl.BlockSpec((8, S, S), lambda og: (og, 0, 0)),
                out_shape=jax.ShapeDtypeStruct((16, S, S), BF16),
            )(a1b, w1e, c1br))

    c2parts = []
    for ip in range(2):
        for jp in range(2):
            for cg in range(2):
                c2parts.append(pl.pallas_call(
                    _make_k3_body(ip, jp),
                    grid=(1,),
                    in_specs=[
                        pl.BlockSpec((8, S, S), lambda i, cg=cg: (cg, 0, 0)),
                        pl.BlockSpec((8, S, S), lambda i, cg=cg: (cg, 0, 0)),
                        pl.BlockSpec((8, S, S), lambda i, cg=cg: (cg, 0, 0)),
                        pl.BlockSpec((8, S, S), lambda i, cg=cg: (cg, 0, 0)),
                        pl.BlockSpec((4, 8, 3, 3), lambda i, cg=cg: (0, cg, 0, 0)),
                    ],
                    out_specs=pl.BlockSpec((4, S, S), lambda i: (0, 0, 0)),
                    out_shape=jax.ShapeDtypeStruct((4, S, S), BF16),
                )(p1q[0], p1q[1], p1q[2], p1q[3], c2w))

    p2 = pl.pallas_call(
        _k3b_body,
        grid=(4,),
        in_specs=[pl.BlockSpec((1, S, S), lambda o: (o, 0, 0))] * 8
                 + [pl.BlockSpec(memory_space=pltpu.SMEM)],
        out_specs=pl.BlockSpec((1, S, S), lambda o: (o, 0, 0)),
        out_shape=jax.ShapeDtypeStruct((4, S, S), BF16),
    )(*c2parts, c2b)

    a1v = a1b.reshape(2, 2, 2, 2, S, S)
    t1wr = t1w.transpose(2, 3, 1, 0)
    t2wr = t2w.transpose(2, 3, 1, 0)

    app = pl.pallas_call(
        _k4_body,
        grid=(4,),
        in_specs=[
            pl.BlockSpec((1, 2, 1, 2, S, S), lambda g: (g // 2, 0, g % 2, 0, 0, 0)),
            pl.BlockSpec((4, S, S), lambda g: (0, 0, 0)),
            pl.BlockSpec((1, 1, 16, 4), lambda g: (g // 2, g % 2, 0, 0)),
            pl.BlockSpec((16,), lambda g: (0,)),
            pl.BlockSpec((2, 2, 3, 16), lambda g: (0, 0, 0, 0)),
            pl.BlockSpec((3,), lambda g: (0,)),
        ],
        out_specs=pl.BlockSpec((3, 1, 1, 2, 2, S, S),
                               lambda g: (0, g // 2, g % 2, 0, 0, 0, 0)),
        out_shape=jax.ShapeDtypeStruct((3, 2, 2, 2, 2, S, S), BF16),
    )(a1v, p2, t1wr, t1b, t2wr, t2b)

    bpp = pl.pallas_call(
        _k5_body,
        grid=(4, 4),
        in_specs=[
            pl.BlockSpec((1, 2, 1, 2, S, S),
                         lambda i, j: (i // 2, 0, i % 2, 0, 0, 0)),
            pl.BlockSpec((2, 1, 2, 1, S, S),
                         lambda i, j: (0, j // 2, 0, j % 2, 0, 0)),
        ],
        out_specs=pl.BlockSpec((1, 1, S, S), lambda i, j: (i, j, 0, 0)),
        out_shape=jax.ShapeDtypeStruct((4, 4, S, S), BF16),
    )(app[0], app[1])

    am, parts = pl.pallas_call(
        _k6_body,
        grid=(4, 4),
        in_specs=[
            pl.BlockSpec((1, 4, S, S), lambda i, j: (i, 0, 0, 0)),
            pl.BlockSpec((2, 1, 2, 1, S, S),
                         lambda i, j: (0, j // 2, 0, j % 2, 0, 0)),
        ],
        out_specs=[
            pl.BlockSpec((1, 1, S, S), lambda i, j: (i, j, 0, 0)),
            pl.BlockSpec((1, 1, S), lambda i, j: (j * 4 + i, 0, 0)),
        ],
        out_shape=[
            jax.ShapeDtypeStruct((4, 4, S, S), BF16),
            jax.ShapeDtypeStruct((16, 1, S), F32),
        ],
    )(bpp, app[2])

    xp = x.reshape(S, 4, x.shape[1]).transpose(1, 0, 2)

    hp = pl.pallas_call(
        _k7_body,
        grid=(4,),
        in_specs=[
            pl.BlockSpec((4, 1, S, S), lambda i: (0, i, 0, 0)),
            pl.BlockSpec((16, 1, S), lambda i: (0, 0, 0)),
            pl.BlockSpec((4, S, 128), lambda i: (0, 0, 0)),
            pl.BlockSpec((128, 64), lambda i: (0, 0)),
            pl.BlockSpec((64,), lambda i: (0,)),
        ],
        out_specs=pl.BlockSpec((1, S, 64), lambda i: (i, 0, 0)),
        out_shape=jax.ShapeDtypeStruct((4, S, 64), F32),
    )(am, parts, xp, W13, b13)

    op = pl.pallas_call(
        _k8_body,
        grid=(4,),
        in_specs=[
            pl.BlockSpec((4, 1, S, S), lambda i: (0, i, 0, 0)),
            pl.BlockSpec((16, 1, S), lambda i: (0, 0, 0)),
            pl.BlockSpec((4, S, 64), lambda i: (0, 0, 0)),
            pl.BlockSpec((64, 16), lambda i: (0, 0)),
            pl.BlockSpec((16,), lambda i: (0,)),
        ],
        out_specs=pl.BlockSpec((1, S, 16), lambda i: (i, 0, 0)),
        out_shape=jax.ShapeDtypeStruct((4, S, 16), F32),
    )(am, parts, hp, W23, b23)

    return op.transpose(1, 0, 2).reshape(N, 16)


# speedup vs baseline: 1.3186x; 1.0003x over previous
import functools

import jax
import jax.numpy as jnp
from jax import lax
from jax.experimental import pallas as pl
from jax.experimental.pallas import tpu as pltpu
from jax.experimental.pallas import tpu_sc as plsc

N = 2048
E = 32768
S = 512
F32 = jnp.float32
BF16 = jnp.bfloat16


def _build_a1pp(edge_index):
    EPT = E // 16
    G = EPT // 16
    QW = 16 * 128 * 512
    mesh = plsc.VectorSubcoreMesh(core_axis_name="c", subcore_axis_name="s")

    @functools.partial(
        pl.kernel,
        mesh=mesh,
        compiler_params=pltpu.CompilerParams(needs_layout_passes=False),
        out_type=jax.ShapeDtypeStruct((16, S * S), F32),
        scratch_types=[
            pltpu.VMEM((EPT,), jnp.int32),
            pltpu.VMEM((EPT,), jnp.int32),
            pltpu.VMEM((2, 16, 128), jnp.int32),
            pltpu.VMEM((2, 16, 128), F32),
            pltpu.VMEM((16384,), F32),
            pltpu.VMEM_SHARED((QW + 8,), F32),
        ],
    )
    def k(edges_hbm, out_hbm, src_v, dst_v, idx_st, val_st, zbuf, shared):
        sc = lax.axis_index("c")
        tile = lax.axis_index("s")
        base_e = tile * EPT
        pltpu.sync_copy(edges_hbm.at[0, pl.ds(base_e, EPT)], src_v)
        pltpu.sync_copy(edges_hbm.at[1, pl.ds(base_e, EPT)], dst_v)

        zeros16 = jnp.zeros((16,), F32)

        def zb_body(i, _):
            zbuf[pl.ds(i * 16, 16)] = zeros16
            return 0
        lax.fori_loop(0, 16384 // 16, zb_body, 0)

        row0 = sc * 1024

        def gbody(g, _):
            s16 = src_v[pl.ds(g * 16, 16)]
            d16 = dst_v[pl.ds(g * 16, 16)]
            for r in range(2):
                lo = row0 + r * 512
                inq = (s16 >= lo) & (s16 < lo + 512)
                addr = ((s16 & 3) * 4 + (d16 & 3)) * 65536                     + ((s16 - lo) >> 2) * 512 + (d16 >> 2)
                idx_st[r, g >> 3, pl.ds((g & 7) * 16, 16)] =                     jnp.where(inq, addr, QW)
                val_st[r, g >> 3, pl.ds((g & 7) * 16, 16)] =                     jnp.where(inq, 1.0, 0.0)
            return 0
        lax.fori_loop(0, G, gbody, 0)

        for r in range(2):
            for z in range(4):
                pltpu.sync_copy(
                    zbuf, shared.at[pl.ds(tile * 65536 + z * 16384, 16384)])
            plsc.subcore_barrier()
            for j in range(16):
                pltpu.sync_copy(val_st.at[r, j],
                                shared.at[idx_st.at[r, j]], add=True)
            plsc.subcore_barrier()
            q = sc * 2 + r
            pltpu.sync_copy(shared.at[pl.ds(tile * 65536, 65536)],
                            out_hbm.at[tile, pl.ds(q * 65536, 65536)])
            plsc.subcore_barrier()

    return k(edge_index)


def _shift2(b, dr, dc):
    z = jnp.zeros_like(b)
    if dr == -1:
        b = jnp.concatenate([z[..., :1, :], b[..., :-1, :]], axis=-2)
    elif dr == 1:
        b = jnp.concatenate([b[..., 1:, :], z[..., :1, :]], axis=-2)
    if dc == -1:
        b = jnp.concatenate([z[..., :, :1], b[..., :, :-1]], axis=-1)
    elif dc == 1:
        b = jnp.concatenate([b[..., :, 1:], z[..., :, :1]], axis=-1)
    return b


def _make_k2_body(ipp, jpp):
    def body(a_ref, w_ref, b_ref, out_ref):
        wmat = w_ref[0].astype(BF16)
        bvec = b_ref[0, 0]
        mx = None
        for di in range(2):
            for dj in range(2):
                ip = 2 * ipp + di
                jp = 2 * jpp + dj
                planes = []
                for a in (-1, 0, 1):
                    q, rI = (ip + a) % 4, (ip + a) // 4
                    for b in (-1, 0, 1):
                        r, rJ = (jp + b) % 4, (jp + b) // 4
                        planes.append(_shift2(a_ref[q * 4 + r], rI, rJ))
                rhs = jnp.stack(planes).reshape(9, S * S)
                res = jnp.dot(wmat, rhs, preferred_element_type=F32)
                v = jax.nn.relu(res + bvec[:, None])
                mx = v if mx is None else jnp.maximum(mx, v)
        out_ref[...] = mx.reshape(8, S, S).astype(BF16)
    return body


def _make_k3_body(ip, jp):
    def body(p00_ref, p01_ref, p10_ref, p11_ref, w_ref, out_ref):
        prefs = ((p00_ref, p01_ref), (p10_ref, p11_ref))
        acc = jnp.zeros((4, S, S), F32)
        for a in (-1, 0, 1):
            q, rI = (ip + a) % 2, (ip + a) // 2
            for b in (-1, 0, 1):
                r, rJ = (jp + b) % 2, (jp + b) // 2
                P = prefs[q][r][...].reshape(8, S * S)
                W = w_ref[:, :, a + 1, b + 1].astype(BF16)
                res = jnp.dot(W, P, preferred_element_type=F32)
                acc = acc + _shift2(res.reshape(4, S, S), rI, rJ)
        out_ref[...] = acc.astype(BF16)
    return body


def _k3b_body(*refs):
    parts = refs[:8]
    bias_ref = refs[8]
    p2_ref = refs[9]
    o2 = pl.program_id(0)
    mx = None
    for par in range(4):
        v = parts[par * 2][0].astype(F32) + parts[par * 2 + 1][0].astype(F32)
        mx = v if mx is None else jnp.maximum(mx, v)
    p2_ref[0] = jax.nn.relu(mx + bias_ref[o2]).astype(BF16)


def _k4_body(a1_ref, p2_ref, t1w_ref, t1b_ref, t2w_ref, t2b_ref, out_ref):
    p2f = p2_ref[...].reshape(4, S * S)
    w1 = t1w_ref[0, 0].astype(BF16)
    u = jnp.dot(w1, p2f, preferred_element_type=F32)
    u = jax.nn.relu(u + t1b_ref[...][:, None]).astype(BF16)
    for qa in range(2):
        for qb in range(2):
            w2 = t2w_ref[qa, qb].astype(BF16)
            att = jnp.dot(w2, u, preferred_element_type=F32)
            att = jax.nn.sigmoid(att + t2b_ref[...][:, None])
            a1 = a1_ref[0, qa, 0, qb].reshape(1, S * S).astype(F32)
            out_ref[:, 0, 0, qa, qb] = (a1 * att).reshape(3, S, S).astype(BF16)


def _k5_body(a0_ref, a1_ref, out_ref):
    acc = jnp.zeros((S, S), F32)
    for kp in range(4):
        acc = acc + jnp.dot(a0_ref[0, kp // 2, 0, kp % 2],
                            a1_ref[kp // 2, 0, kp % 2, 0],
                            preferred_element_type=F32)
    out_ref[0, 0] = acc.astype(BF16)


def _k6_body(b_ref, a2_ref, am_ref, parts_ref):
    ip = pl.program_id(0)
    jp = pl.program_id(1)
    acc = jnp.zeros((S, S), F32)
    for kp in range(4):
        acc = acc + jnp.dot(b_ref[0, kp],
                            a2_ref[kp // 2, 0, kp % 2, 0],
                            preferred_element_type=F32)
    pos = acc > 0.0
    am = jnp.where(pos, jnp.exp(jnp.log(jnp.where(pos, acc, 1.0)) * (1.0 / 3.0)), 0.0)
    rows = lax.broadcasted_iota(jnp.int32, (S, S), 0)
    cols = lax.broadcasted_iota(jnp.int32, (S, S), 1)
    eye = (rows == cols) & (ip == jp)
    am = jnp.where(eye & (am == 0.0), 1.0, am)
    am_ref[0, 0] = am.astype(BF16)
    parts_ref[0, 0] = jnp.sum(am, axis=0)


def _dinv_from_parts(parts):
    dinv = []
    for jp in range(4):
        deg = parts[jp * 4 + 0, 0] + parts[jp * 4 + 1, 0] + \
              parts[jp * 4 + 2, 0] + parts[jp * 4 + 3, 0]
        dinv.append(jnp.where(deg > 0, jax.lax.rsqrt(jnp.where(deg > 0, deg, 1.0)), 0.0))
    return dinv


def _select_row(rows, idx):
    out = jnp.zeros((S,), F32)
    for j in range(4):
        out = out + jnp.where(idx == j, 1.0, 0.0) * rows[j]
    return out


def _k7_body(am_ref, parts_ref, xp_ref, w_ref, b_ref, out_ref):
    ip = pl.program_id(0)
    dinv = _dinv_from_parts(parts_ref[...])
    acc = jnp.zeros((S, w_ref.shape[1]), F32)
    for jp in range(4):
        z = dinv[jp][:, None] * jnp.dot(xp_ref[jp], w_ref[...],
                                        preferred_element_type=F32)
        acc = acc + lax.dot_general(am_ref[jp, 0], z.astype(BF16),
                                    (((0,), (0,)), ((), ())),
                                    preferred_element_type=F32)
    dout = _select_row(dinv, ip)
    out_ref[0] = jax.nn.relu(dout[:, None] * acc + b_ref[...][None, :])


def _k8_body(am_ref, parts_ref, hp_ref, w_ref, b_ref, out_ref):
    ip = pl.program_id(0)
    dinv = _dinv_from_parts(parts_ref[...])
    acc = jnp.zeros((S, w_ref.shape[1]), F32)
    for jp in range(4):
        z = dinv[jp][:, None] * jnp.dot(hp_ref[jp], w_ref[...],
                                        preferred_element_type=F32)
        acc = acc + lax.dot_general(am_ref[jp, 0], z.astype(BF16),
                                    (((0,), (0,)), ((), ())),
                                    preferred_element_type=F32)
    dout = _select_row(dinv, ip)
    o = dout[:, None] * acc + b_ref[...][None, :]
    m = jnp.max(o, axis=1, keepdims=True)
    s = jnp.log(jnp.sum(jnp.exp(o - m), axis=1, keepdims=True))
    out_ref[0] = o - m - s


def kernel(x, c1w, c1b, c2w, c2b, t1w, t1b, t2w, t2b,
           W11, b11, W13, b13, W23, b23, edge_index):
    del W11, b11

    a1pp = _build_a1pp(edge_index).reshape(16, S, S)
    a1b = a1pp.astype(BF16)

    w1e = c1w.sum(axis=1).reshape(2, 8, 9)
    c1br = c1b.reshape(2, 1, 8)

    p1q = []
    for ipp in range(2):
        for jpp in range(2):
            p1q.append(pl.pallas_call(
                _make_k2_body(ipp, jpp),
                grid=(2,),
                in_specs=[
                    pl.BlockSpec((16, S, S), lambda og: (0, 0, 0)),
                    pl.BlockSpec((1, 8, 9), lambda og: (og, 0, 0)),
                    pl.BlockSpec((1, 1, 8), lambda og: (og, 0, 0)),
                ],
                out_specs=pl.BlockSpec((8, S, S), lambda og: (og, 0, 0)),
                out_shape=jax.ShapeDtypeStruct((16, S, S), BF16),
            )(a1b, w1e, c1br))

    c2parts = []
    for ip in range(2):
        for jp in range(2):
            for cg in range(2):
                c2parts.append(pl.pallas_call(
                    _make_k3_body(ip, jp),
                    grid=(1,),
                    in_specs=[
                        pl.BlockSpec((8, S, S), lambda i, cg=cg: (cg, 0, 0)),
                        pl.BlockSpec((8, S, S), lambda i, cg=cg: (cg, 0, 0)),
                        pl.BlockSpec((8, S, S), lambda i, cg=cg: (cg, 0, 0)),
                        pl.BlockSpec((8, S, S), lambda i, cg=cg: (cg, 0, 0)),
                        pl.BlockSpec((4, 8, 3, 3), lambda i, cg=cg: (0, cg, 0, 0)),
                    ],
                    out_specs=pl.BlockSpec((4, S, S), lambda i: (0, 0, 0)),
                    out_shape=jax.ShapeDtypeStruct((4, S, S), BF16),
                )(p1q[0], p1q[1], p1q[2], p1q[3], c2w))

    p2 = pl.pallas_call(
        _k3b_body,
        grid=(4,),
        in_specs=[pl.BlockSpec((1, S, S), lambda o: (o, 0, 0))] * 8
                 + [pl.BlockSpec(memory_space=pltpu.SMEM)],
        out_specs=pl.BlockSpec((1, S, S), lambda o: (o, 0, 0)),
        out_shape=jax.ShapeDtypeStruct((4, S, S), BF16),
    )(*c2parts, c2b)

    a1v = a1b.reshape(2, 2, 2, 2, S, S)
    t1wr = t1w.transpose(2, 3, 1, 0)
    t2wr = t2w.transpose(2, 3, 1, 0)

    app = pl.pallas_call(
        _k4_body,
        grid=(4,),
        in_specs=[
            pl.BlockSpec((1, 2, 1, 2, S, S), lambda g: (g // 2, 0, g % 2, 0, 0, 0)),
            pl.BlockSpec((4, S, S), lambda g: (0, 0, 0)),
            pl.BlockSpec((1, 1, 16, 4), lambda g: (g // 2, g % 2, 0, 0)),
            pl.BlockSpec((16,), lambda g: (0,)),
            pl.BlockSpec((2, 2, 3, 16), lambda g: (0, 0, 0, 0)),
            pl.BlockSpec((3,), lambda g: (0,)),
        ],
        out_specs=pl.BlockSpec((3, 1, 1, 2, 2, S, S),
                               lambda g: (0, g // 2, g % 2, 0, 0, 0, 0)),
        out_shape=jax.ShapeDtypeStruct((3, 2, 2, 2, 2, S, S), BF16),
    )(a1v, p2, t1wr, t1b, t2wr, t2b)

    bpp = pl.pallas_call(
        _k5_body,
        grid=(4, 4),
        in_specs=[
            pl.BlockSpec((1, 2, 1, 2, S, S),
                         lambda i, j: (i // 2, 0, i % 2, 0, 0, 0)),
            pl.BlockSpec((2, 1, 2, 1, S, S),
                         lambda i, j: (0, j // 2, 0, j % 2, 0, 0)),
        ],
        out_specs=pl.BlockSpec((1, 1, S, S), lambda i, j: (i, j, 0, 0)),
        out_shape=jax.ShapeDtypeStruct((4, 4, S, S), BF16),
    )(app[0], app[1])

    am, parts = pl.pallas_call(
        _k6_body,
        grid=(4, 4),
        in_specs=[
            pl.BlockSpec((1, 4, S, S), lambda i, j: (i, 0, 0, 0)),
            pl.BlockSpec((2, 1, 2, 1, S, S),
                         lambda i, j: (0, j // 2, 0, j % 2, 0, 0)),
        ],
        out_specs=[
            pl.BlockSpec((1, 1, S, S), lambda i, j: (i, j, 0, 0)),
            pl.BlockSpec((1, 1, S), lambda i, j: (j * 4 + i, 0, 0)),
        ],
        out_shape=[
            jax.ShapeDtypeStruct((4, 4, S, S), BF16),
            jax.ShapeDtypeStruct((16, 1, S), F32),
        ],
    )(bpp, app[2])

    xp = x.reshape(S, 4, x.shape[1]).transpose(1, 0, 2)

    hp = pl.pallas_call(
        _k7_body,
        grid=(4,),
        in_specs=[
            pl.BlockSpec((4, 1, S, S), lambda i: (0, i, 0, 0)),
            pl.BlockSpec((16, 1, S), lambda i: (0, 0, 0)),
            pl.BlockSpec((4, S, 128), lambda i: (0, 0, 0)),
            pl.BlockSpec((128, 64), lambda i: (0, 0)),
            pl.BlockSpec((64,), lambda i: (0,)),
        ],
        out_specs=pl.BlockSpec((1, S, 64), lambda i: (i, 0, 0)),
        out_shape=jax.ShapeDtypeStruct((4, S, 64), F32),
    )(am, parts, xp, W13, b13)

    op = pl.pallas_call(
        _k8_body,
        grid=(4,),
        in_specs=[
            pl.BlockSpec((4, 1, S, S), lambda i: (0, i, 0, 0)),
            pl.BlockSpec((16, 1, S), lambda i: (0, 0, 0)),
            pl.BlockSpec((4, S, 64), lambda i: (0, 0, 0)),
            pl.BlockSpec((64, 16), lambda i: (0, 0)),
            pl.BlockSpec((16,), lambda i: (0,)),
        ],
        out_specs=pl.BlockSpec((1, S, 16), lambda i: (i, 0, 0)),
        out_shape=jax.ShapeDtypeStruct((4, S, 16), F32),
    )(am, parts, hp, W23, b23)

    return op.transpose(1, 0, 2).reshape(N, 16)
